# baseline (speedup 1.0000x reference)
import functools

import jax
import jax.numpy as jnp
from jax import lax
from jax.experimental import pallas as pl
from jax.experimental.pallas import tpu as pltpu
from jax.experimental.pallas import tpu_sc as plsc

NC, NS, L = 2, 16, 16
NW = NC * NS
B, T, D = 16384, 50, 32
BAG_PAD = 128
BAGS_PER_W = B // NW
T_G = 56
PAIR_W = 2 * T_G
CHUNK = 256
PAIRS = CHUNK // 2
NBUF = 8


def _embedding_bag_sc(idx_pad, emb_table):
    mesh = plsc.VectorSubcoreMesh(core_axis_name="c", subcore_axis_name="s")

    @functools.partial(
        pl.kernel,
        mesh=mesh,
        out_type=jax.ShapeDtypeStruct((B * D,), jnp.float32),
        compiler_params=pltpu.CompilerParams(use_tc_tiling_on_sc=False),
        scratch_types=[
            pltpu.VMEM((CHUNK, BAG_PAD), jnp.int32),
            pltpu.VMEM((PAIRS, BAG_PAD), jnp.int32),
            pltpu.VMEM((NBUF, BAG_PAD, D), jnp.float32),
            pltpu.VMEM((BAGS_PER_W * D,), jnp.float32),
            pltpu.SemaphoreType.DMA((NBUF,)),
        ],
    )
    def bag_kernel(idx_hbm, table_hbm, out_hbm, idx_v, idx_c, rows_v, out_v, sem):
        wid = lax.axis_index("s") * NC + lax.axis_index("c")
        zeros16 = jnp.zeros((L,), jnp.int32)

        for chunk in range(BAGS_PER_W // CHUNK):
            base = wid * BAGS_PER_W + chunk * CHUNK
            pltpu.sync_copy(idx_hbm.at[pl.ds(base, CHUNK)], idx_v)

            @pl.loop(0, PAIRS)
            def _(k):
                for off in (0, 16, 32, 40):
                    idx_c[k, pl.ds(off, L)] = idx_v[2 * k, pl.ds(off, L)]
                    idx_c[k, pl.ds(T_G + off, L)] = idx_v[2 * k + 1, pl.ds(off, L)]
                idx_c[k, pl.ds(PAIR_W, L)] = zeros16

            for b in range(NBUF):
                pltpu.make_async_copy(
                    table_hbm.at[idx_c.at[b]],
                    rows_v.at[b], sem.at[b]).start()

            @pl.loop(0, PAIRS, step=NBUF)
            def _(j0):
                for b in range(NBUF):
                    j = j0 + b
                    buf = rows_v.at[b]
                    pltpu.make_async_copy(
                        table_hbm.at[idx_c.at[j]],
                        buf, sem.at[b]).wait()
                    for bag in range(2):
                        r0 = bag * T_G
                        for h in range(D // L):
                            acc0 = buf[r0, pl.ds(h * L, L)]
                            acc1 = buf[r0 + 1, pl.ds(h * L, L)]
                            for r in range(2, T, 2):
                                acc0 = acc0 + buf[r0 + r, pl.ds(h * L, L)]
                                acc1 = acc1 + buf[r0 + r + 1, pl.ds(h * L, L)]
                            off = (chunk * CHUNK + 2 * j + bag) * D + h * L
                            out_v[pl.ds(off, L)] = acc0 + acc1

                    @pl.when(j + NBUF < PAIRS)
                    def _():
                        pltpu.make_async_copy(
                            table_hbm.at[idx_c.at[j + NBUF]],
                            buf, sem.at[b]).start()

        pltpu.sync_copy(out_v, out_hbm.at[pl.ds(wid * BAGS_PER_W * D, BAGS_PER_W * D)])

    return bag_kernel(idx_pad, emb_table)


BK = 2048


def _mlp_body(x_ref, e_ref, w1a, w1b, b1r, g1r, be1r, w2, b2r, g2r, be2r, w3, b3r, o_ref):
    h = jnp.dot(x_ref[...], w1a[...], preferred_element_type=jnp.float32)
    h = h + jnp.dot(e_ref[...], w1b[...], preferred_element_type=jnp.float32)
    h = h + b1r[...]
    h = jnp.maximum(h, 0.0)
    mu = jnp.mean(h, axis=-1, keepdims=True)
    var = jnp.mean((h - mu) ** 2, axis=-1, keepdims=True)
    h = (h - mu) / jnp.sqrt(var + 1e-5) * g1r[...] + be1r[...]
    h = jnp.dot(h, w2[...], preferred_element_type=jnp.float32) + b2r[...]
    h = jnp.maximum(h, 0.0)
    mu = jnp.mean(h, axis=-1, keepdims=True)
    var = jnp.mean((h - mu) ** 2, axis=-1, keepdims=True)
    h = (h - mu) / jnp.sqrt(var + 1e-5) * g2r[...] + be2r[...]
    o_ref[...] = jnp.dot(h, w3[...], preferred_element_type=jnp.float32) + b3r[...]


def _mlp_tc(x_num, emb, W1a, W1b, b1, g1, be1, W2, b2, g2, be2, W3p, b3p):
    n_feat = x_num.shape[1]
    full = lambda a: pl.BlockSpec(a.shape, lambda i: (0, 0))
    return pl.pallas_call(
        _mlp_body,
        grid=(B // BK,),
        in_specs=[
            pl.BlockSpec((BK, n_feat), lambda i: (i, 0)),
            pl.BlockSpec((BK, D), lambda i: (i, 0)),
            full(W1a), full(W1b), full(b1), full(g1), full(be1),
            full(W2), full(b2), full(g2), full(be2),
            full(W3p), full(b3p),
        ],
        out_specs=pl.BlockSpec((BK, 8), lambda i: (i, 0)),
        out_shape=jax.ShapeDtypeStruct((B, 8), jnp.float32),
    )(x_num, emb, W1a, W1b, b1, g1, be1, W2, b2, g2, be2, W3p, b3p)


def kernel(x_num, leaf_ids, emb_table, W1, b1, g1, be1, W2, b2, g2, be2, W3, b3):
    idx_pad = jnp.pad(leaf_ids.astype(jnp.int32), ((0, 0), (0, BAG_PAD - T)))
    emb_flat = _embedding_bag_sc(idx_pad, emb_table)
    emb = emb_flat.reshape(B, D)

    n_feat = x_num.shape[1]
    W1a, W1b = W1[:n_feat], W1[n_feat:]
    W3p = jnp.zeros((W3.shape[0], 8), jnp.float32).at[:, :3].set(W3)
    b3p = jnp.zeros((8,), jnp.float32).at[:3].set(b3)

    out = _mlp_tc(
        x_num, emb, W1a, W1b,
        b1.reshape(1, -1), g1.reshape(1, -1), be1.reshape(1, -1),
        W2, b2.reshape(1, -1), g2.reshape(1, -1), be2.reshape(1, -1),
        W3p, b3p.reshape(1, -1),
    )
    return out[:, :3]


# speedup vs baseline: 6.2665x; 6.2665x over previous
import functools

import jax
import jax.numpy as jnp
from jax import lax
from jax.experimental import pallas as pl
from jax.experimental.pallas import tpu as pltpu
from jax.experimental.pallas import tpu_sc as plsc

NC, NS, L = 2, 16, 16
NW = NC * NS
B, T, D = 16384, 50, 32
V = 1000000
BAGS_PER_STEP = 2
ROWS_PER_STEP = BAGS_PER_STEP * T
BAGS_PER_W = B // NW
STEPS = BAGS_PER_W // BAGS_PER_STEP
NBUF = 8


def _embedding_bag_sc(idx2d, emb_table):
    mesh = plsc.VectorSubcoreMesh(core_axis_name="c", subcore_axis_name="s")

    @functools.partial(
        pl.kernel,
        mesh=mesh,
        out_type=jax.ShapeDtypeStruct((B * D,), jnp.float32),
        compiler_params=pltpu.CompilerParams(use_tc_tiling_on_sc=False),
        scratch_types=[
            pltpu.VMEM((STEPS, ROWS_PER_STEP), jnp.int32),
            pltpu.VMEM((NBUF, ROWS_PER_STEP, D), jnp.float32),
            pltpu.VMEM((BAGS_PER_W * D,), jnp.float32),
            pltpu.SemaphoreType.DMA((NBUF,)),
        ],
    )
    def bag_kernel(idx_hbm, table_hbm, out_hbm, idx_v, rows_v, out_v, sem):
        wid = lax.axis_index("s") * NC + lax.axis_index("c")
        pltpu.sync_copy(idx_hbm.at[pl.ds(wid * STEPS, STEPS)], idx_v)

        for b in range(NBUF):
            pltpu.make_async_copy(
                table_hbm.at[idx_v.at[b]], rows_v.at[b], sem.at[b]).start()

        @pl.loop(0, STEPS, step=NBUF)
        def _(j0):
            for b in range(NBUF):
                j = j0 + b
                buf = rows_v.at[b]
                pltpu.make_async_copy(
                    table_hbm.at[idx_v.at[j]], buf, sem.at[b]).wait()
                for bag in range(BAGS_PER_STEP):
                    for h in range(D // L):
                        acc0 = buf[bag * T, pl.ds(h * L, L)]
                        acc1 = buf[bag * T + 1, pl.ds(h * L, L)]
                        for r in range(2, T, 2):
                            acc0 = acc0 + buf[bag * T + r, pl.ds(h * L, L)]
                            acc1 = acc1 + buf[bag * T + r + 1, pl.ds(h * L, L)]
                        off = (j * BAGS_PER_STEP + bag) * D + h * L
                        out_v[pl.ds(off, L)] = acc0 + acc1

                @pl.when(j + NBUF < STEPS)
                def _():
                    pltpu.make_async_copy(
                        table_hbm.at[idx_v.at[j + NBUF]], buf, sem.at[b]).start()

        pltpu.sync_copy(out_v, out_hbm.at[pl.ds(wid * BAGS_PER_W * D, BAGS_PER_W * D)])

    return bag_kernel(idx2d, emb_table)


TCH = 4096
QR = TCH // 4
NBLK = (V + TCH - 1) // TCH
VP = NBLK * TCH


def _relayout_body(t_ref, o_ref):
    for k in range(4):
        o_ref[:, k * D:(k + 1) * D] = jnp.transpose(
            t_ref[:, k * QR:(k + 1) * QR])


def _relayout_table(tbl_t):
    return pl.pallas_call(
        _relayout_body,
        grid=(NBLK,),
        in_specs=[pl.BlockSpec((D, TCH), lambda i: (0, i))],
        out_specs=pl.BlockSpec((QR, 128), lambda i: (i, 0)),
        out_shape=jax.ShapeDtypeStruct((VP // 4, 128), jnp.float32),
    )(tbl_t)


BK = 2048


def _mlp_body(x_ref, e_ref, w1a, w1b, b1r, g1r, be1r, w2, b2r, g2r, be2r, w3, b3r, o_ref):
    h = jnp.dot(x_ref[...], w1a[...], preferred_element_type=jnp.float32)
    h = h + jnp.dot(e_ref[...], w1b[...], preferred_element_type=jnp.float32)
    h = h + b1r[...]
    h = jnp.maximum(h, 0.0)
    mu = jnp.mean(h, axis=-1, keepdims=True)
    var = jnp.mean((h - mu) ** 2, axis=-1, keepdims=True)
    h = (h - mu) / jnp.sqrt(var + 1e-5) * g1r[...] + be1r[...]
    h = jnp.dot(h, w2[...], preferred_element_type=jnp.float32) + b2r[...]
    h = jnp.maximum(h, 0.0)
    mu = jnp.mean(h, axis=-1, keepdims=True)
    var = jnp.mean((h - mu) ** 2, axis=-1, keepdims=True)
    h = (h - mu) / jnp.sqrt(var + 1e-5) * g2r[...] + be2r[...]
    o_ref[...] = jnp.dot(h, w3[...], preferred_element_type=jnp.float32) + b3r[...]


def _mlp_tc(x_num, emb, W1a, W1b, b1, g1, be1, W2, b2, g2, be2, W3p, b3p):
    n_feat = x_num.shape[1]
    full = lambda a: pl.BlockSpec(a.shape, lambda i: (0, 0))
    return pl.pallas_call(
        _mlp_body,
        grid=(B // BK,),
        in_specs=[
            pl.BlockSpec((BK, n_feat), lambda i: (i, 0)),
            pl.BlockSpec((BK, D), lambda i: (i, 0)),
            full(W1a), full(W1b), full(b1), full(g1), full(be1),
            full(W2), full(b2), full(g2), full(be2),
            full(W3p), full(b3p),
        ],
        out_specs=pl.BlockSpec((BK, 8), lambda i: (i, 0)),
        out_shape=jax.ShapeDtypeStruct((B, 8), jnp.float32),
    )(x_num, emb, W1a, W1b, b1, g1, be1, W2, b2, g2, be2, W3p, b3p)


def kernel(x_num, leaf_ids, emb_table, W1, b1, g1, be1, W2, b2, g2, be2, W3, b3):
    idx = leaf_ids.astype(jnp.int32).reshape(B * T // ROWS_PER_STEP, ROWS_PER_STEP)
    idx2d = (idx & ~(TCH - 1)) + ((idx & (QR - 1)) << 2) + ((idx & (TCH - 1)) >> 10)
    tbl = _relayout_table(emb_table.T).reshape(VP, D)
    emb_flat = _embedding_bag_sc(idx2d, tbl)
    emb = emb_flat.reshape(B, D)

    n_feat = x_num.shape[1]
    W1a, W1b = W1[:n_feat], W1[n_feat:]
    W3p = jnp.zeros((W3.shape[0], 8), jnp.float32).at[:, :3].set(W3)
    b3p = jnp.zeros((8,), jnp.float32).at[:3].set(b3)

    out = _mlp_tc(
        x_num, emb, W1a, W1b,
        b1.reshape(1, -1), g1.reshape(1, -1), be1.reshape(1, -1),
        W2, b2.reshape(1, -1), g2.reshape(1, -1), be2.reshape(1, -1),
        W3p, b3p.reshape(1, -1),
    )
    return out[:, :3]
